# Initial kernel scaffold; baseline (speedup 1.0000x reference)
import jax
import jax.numpy as jnp
from jax.experimental import pallas as pl


def kernel(weight_padded, indices, offsets, valid_count):
    raise NotImplementedError("write your pallas kernel here")


# speedup vs baseline: 7.6362x; 7.6362x over previous
import functools

import jax
import jax.numpy as jnp
from jax import lax
from jax.experimental import pallas as pl
from jax.experimental.pallas import tpu as pltpu


def _gather_body(tl, unroll, idx_s, w_ref, p_ref):
    base = pl.program_id(0) * tl

    def chunk(c, carry):
        j = c * unroll
        for u in range(unroll):
            p_ref[j + u] = w_ref[idx_s[base + j + u]]
        return carry

    lax.fori_loop(0, tl // unroll, chunk, 0)


def _seg_body(tb, tk, lo_ref, hi_ref, p_ref, out_ref):
    k = pl.program_id(1)

    @pl.when(k == 0)
    def _():
        out_ref[...] = jnp.zeros_like(out_ref)

    pos = k * tk + lax.broadcasted_iota(jnp.int32, (tb, tk), 1)
    a = jnp.logical_and(pos >= lo_ref[...], pos < hi_ref[...]).astype(
        jnp.float32)
    out_ref[...] += jnp.dot(a, p_ref[...], preferred_element_type=jnp.float32)


def _embedding_bag(weight_padded, indices, offsets, valid_count):
    n_pad, m_pad = weight_padded.shape
    L = indices.shape[0]
    num_bags = offsets.shape[0]

    tl = min(512, L)
    unroll = 8
    tb = min(512, num_bags)
    tk = tl

    valid = valid_count.reshape(()).astype(jnp.int32)
    off = offsets.astype(jnp.int32)
    off_ext = jnp.concatenate([off, jnp.full((1,), L, jnp.int32)])
    lo = jnp.minimum(off_ext[:-1], valid).reshape(num_bags, 1)
    hi = jnp.minimum(off_ext[1:], valid).reshape(num_bags, 1)

    idx = indices.astype(jnp.int32)
    w3 = weight_padded.reshape(n_pad, 1, m_pad)

    p = pl.pallas_call(
        functools.partial(_gather_body, tl, unroll),
        out_shape=jax.ShapeDtypeStruct((L, 1, m_pad), jnp.float32),
        grid_spec=pltpu.PrefetchScalarGridSpec(
            num_scalar_prefetch=1,
            grid=(L // tl,),
            in_specs=[
                pl.BlockSpec((n_pad, 1, m_pad), lambda t, s: (0, 0, 0)),
            ],
            out_specs=pl.BlockSpec((tl, 1, m_pad), lambda t, s: (t, 0, 0)),
        ),
        compiler_params=pltpu.CompilerParams(
            dimension_semantics=("parallel",),
            vmem_limit_bytes=40 * 1024 * 1024,
        ),
    )(idx, w3)

    out = pl.pallas_call(
        functools.partial(_seg_body, tb, tk),
        out_shape=jax.ShapeDtypeStruct((num_bags, m_pad), jnp.float32),
        grid=(num_bags // tb, L // tk),
        in_specs=[
            pl.BlockSpec((tb, 1), lambda t, k: (t, 0)),
            pl.BlockSpec((tb, 1), lambda t, k: (t, 0)),
            pl.BlockSpec((tk, m_pad), lambda t, k: (k, 0)),
        ],
        out_specs=pl.BlockSpec((tb, m_pad), lambda t, k: (t, 0)),
        compiler_params=pltpu.CompilerParams(
            dimension_semantics=("parallel", "arbitrary"),
            vmem_limit_bytes=32 * 1024 * 1024,
        ),
    )(lo, hi, p.reshape(L, m_pad))

    return out


def kernel(weight_padded, indices, offsets, valid_count):
    return _embedding_bag(weight_padded, indices, offsets, valid_count)
